# baseline (speedup 1.0000x reference)
import jax
import jax.numpy as jnp
from jax.experimental import pallas as pl
from jax.experimental.pallas import tpu as pltpu
from jax.sharding import Mesh, PartitionSpec as P

_N, _F, _H, _C = 100000, 128, 128, 64
_NDEV = 2
_NSH = _N // _NDEV
_R = 2000
_S = _NSH // _R
_NBUF = 5
_GROUPS = _S // _NBUF


def _mlp_kernel(x_hbm, w1_ref, b1_ref, w2_ref, b2_ref, out_hbm, *scratch):
    xbufs = scratch[:_NBUF]
    obufs = scratch[_NBUF:2 * _NBUF]
    in_sem = scratch[2 * _NBUF]
    out_sem = scratch[2 * _NBUF + 1]

    def in_copy(c, k):
        return pltpu.make_async_copy(
            x_hbm.at[pl.ds(c * _R, _R)], xbufs[k], in_sem.at[k])

    def out_copy(c, k):
        return pltpu.make_async_copy(
            obufs[k], out_hbm.at[pl.ds(c * _R, _R)], out_sem.at[k])

    for k in range(_NBUF):
        in_copy(k, k).start()

    w1 = w1_ref[...]
    b1 = b1_ref[...]
    w2 = w2_ref[...]
    b2 = b2_ref[...]

    def group(i, carry):
        for k in range(_NBUF):
            c = i * _NBUF + k
            in_copy(c, k).wait()

            @pl.when(i >= 1)
            def _():
                out_copy(c - _NBUF, k).wait()

            h = jax.lax.dot_general(
                xbufs[k][...], w1,
                dimension_numbers=(((1,), (1,)), ((), ())),
                preferred_element_type=jnp.float32)
            h = jnp.maximum(h + b1, 0.0)
            obufs[k][...] = jax.lax.dot_general(
                h, w2,
                dimension_numbers=(((1,), (1,)), ((), ())),
                preferred_element_type=jnp.float32) + b2

            out_copy(c, k).start()

            @pl.when(c + _NBUF < _S)
            def _():
                in_copy(c + _NBUF, k).start()
        return carry

    jax.lax.fori_loop(0, _GROUPS, group, 0)

    for k in range(_NBUF):
        out_copy(_S - _NBUF + k, k).wait()


def _per_shard(x_sh, W_in, b1, W_out, b2):
    scratch = (
        [pltpu.VMEM((_R, _F), jnp.float32) for _ in range(_NBUF)]
        + [pltpu.VMEM((_R, _C), jnp.float32) for _ in range(_NBUF)]
        + [pltpu.SemaphoreType.DMA((_NBUF,)),
           pltpu.SemaphoreType.DMA((_NBUF,))]
    )
    return pl.pallas_call(
        _mlp_kernel,
        in_specs=[pl.BlockSpec(memory_space=pltpu.MemorySpace.HBM)]
        + [pl.BlockSpec(memory_space=pltpu.MemorySpace.VMEM)] * 4,
        out_specs=pl.BlockSpec(memory_space=pltpu.MemorySpace.HBM),
        out_shape=jax.ShapeDtypeStruct((_NSH, _C), jnp.float32),
        scratch_shapes=scratch,
    )(x_sh, W_in, b1, W_out, b2)


def kernel(nodeblocks, x, W_in, b_in, W_out, b_out):
    b1 = b_in.reshape(1, _H)
    b2 = b_out.reshape(1, _C)
    devs = jax.devices()[:_NDEV]
    if len(devs) < _NDEV:
        halves = [_per_shard(x[i * _NSH:(i + 1) * _NSH], W_in, b1, W_out, b2)
                  for i in range(_NDEV)]
        return jnp.concatenate(halves, axis=0)
    mesh = Mesh(devs, ("i",))
    f = jax.shard_map(
        _per_shard,
        mesh=mesh,
        in_specs=(P("i"), P(), P(), P(), P()),
        out_specs=P("i"),
        check_vma=False,
    )
    return f(x, W_in, b1, W_out, b2)


# speedup vs baseline: 7.3012x; 7.3012x over previous
import jax
import jax.numpy as jnp
from jax.experimental import pallas as pl
from jax.experimental.pallas import tpu as pltpu

_N, _F, _H, _C = 100000, 128, 128, 64
_R = 2000
_S = _N // _R
_NBUF = 10
_GROUPS = _S // _NBUF


def _mlp_kernel(x_hbm, w1_ref, b1_ref, w2_ref, b2_ref, out_hbm, *scratch):
    xbufs = scratch[:_NBUF]
    obufs = scratch[_NBUF:2 * _NBUF]
    in_sem = scratch[2 * _NBUF]
    out_sem = scratch[2 * _NBUF + 1]

    def in_copy(c, k):
        return pltpu.make_async_copy(
            x_hbm.at[pl.ds(c * _R, _R)], xbufs[k], in_sem.at[k])

    def out_copy(c, k):
        return pltpu.make_async_copy(
            obufs[k], out_hbm.at[pl.ds(c * _R, _R)], out_sem.at[k])

    for k in range(_NBUF):
        in_copy(k, k).start()

    w1 = w1_ref[...]
    b1 = b1_ref[...]
    w2 = w2_ref[...]
    b2 = b2_ref[...]

    def group(i, carry):
        for k in range(_NBUF):
            c = i * _NBUF + k
            in_copy(c, k).wait()

            @pl.when(i >= 1)
            def _():
                out_copy(c - _NBUF, k).wait()

            h = jax.lax.dot_general(
                xbufs[k][...], w1,
                dimension_numbers=(((1,), (1,)), ((), ())),
                preferred_element_type=jnp.float32,
            )
            h = jnp.maximum(h + b1, 0.0)
            obufs[k][...] = jax.lax.dot_general(
                h, w2,
                dimension_numbers=(((1,), (1,)), ((), ())),
                preferred_element_type=jnp.float32,
            ) + b2

            out_copy(c, k).start()

            @pl.when(c + _NBUF < _S)
            def _():
                in_copy(c + _NBUF, k).start()
        return carry

    jax.lax.fori_loop(0, _GROUPS, group, 0)

    for k in range(_NBUF):
        out_copy(_S - _NBUF + k, k).wait()


def kernel(nodeblocks, x, W_in, b_in, W_out, b_out):
    b1 = b_in.reshape(1, _H)
    b2 = b_out.reshape(1, _C)
    scratch = (
        [pltpu.VMEM((_R, _F), jnp.float32) for _ in range(_NBUF)]
        + [pltpu.VMEM((_R, _C), jnp.float32) for _ in range(_NBUF)]
        + [pltpu.SemaphoreType.DMA((_NBUF,)),
           pltpu.SemaphoreType.DMA((_NBUF,))]
    )
    return pl.pallas_call(
        _mlp_kernel,
        in_specs=[
            pl.BlockSpec(memory_space=pltpu.MemorySpace.HBM),
            pl.BlockSpec(memory_space=pltpu.MemorySpace.VMEM),
            pl.BlockSpec(memory_space=pltpu.MemorySpace.VMEM),
            pl.BlockSpec(memory_space=pltpu.MemorySpace.VMEM),
            pl.BlockSpec(memory_space=pltpu.MemorySpace.VMEM),
        ],
        out_specs=pl.BlockSpec(memory_space=pltpu.MemorySpace.HBM),
        out_shape=jax.ShapeDtypeStruct((_N, _C), jnp.float32),
        scratch_shapes=scratch,
    )(x, W_in, b1, W_out, b2)
